# baseline (speedup 1.0000x reference)
import jax
import jax.numpy as jnp
import numpy as np
from jax import lax
from jax.experimental import pallas as pl
from jax.experimental.pallas import tpu as pltpu
from jax.experimental.pallas import tpu_sc as plsc

DIM = 33
SHIFT = DIM ** 3
ROW = 35944
NPX = 512 * 512
NBATCH = 4
CHUNK = 1024
L = 16
NWORKERS = 32
PX_PER_W = NBATCH * NPX // NWORKERS
NCHUNKS = PX_PER_W // CHUNK

_INV_BS = np.float32(1.0 / (1.000001 / (DIM - 1)))
_OFFS = (0, 1, DIM, DIM + 1, DIM * DIM, DIM * DIM + 1,
         DIM * DIM + DIM, DIM * DIM + DIM + 1)
_LOAD_BYTES = 3 * CHUNK * 4


def _body(lut_hbm, x_hbm, out_hbm, tbl0, tbl1, tbl2,
          rin0, gin0, bin0, rin1, gin1, bin1,
          rout0, gout0, bout0, rout1, gout1, bout1,
          tsem, lsem0, lsem1, ssem0, ssem1):
    ins = ((rin0, gin0, bin0), (rin1, gin1, bin1))
    outs = ((rout0, gout0, bout0), (rout1, gout1, bout1))
    lsems = (lsem0, lsem1)
    ssems = (ssem0, ssem1)

    nc = lax.axis_size("c")
    wid = lax.axis_index("s") * nc + lax.axis_index("c")

    w_per_batch = NPX // PX_PER_W
    batch = wid // w_per_batch
    plane_off = (wid % w_per_batch) * PX_PER_W
    bases = ((batch * 3 + 0) * NPX + plane_off,
             (batch * 3 + 1) * NPX + plane_off,
             (batch * 3 + 2) * NPX + plane_off)

    tbls = (tbl0, tbl1, tbl2)
    tbl_cps = [pltpu.async_copy(lut_hbm.at[pl.ds(c * ROW, ROW)],
                                tbls[c], tsem) for c in range(3)]

    def issue_loads(t, slot):
        o = t * CHUNK
        for c in range(3):
            pltpu.async_copy(x_hbm.at[pl.ds(bases[c] + o, CHUNK)],
                             ins[slot][c], lsems[slot])

    def wait_loads(slot):
        for c in range(3):
            pltpu.make_async_copy(x_hbm.at[pl.ds(0, CHUNK)],
                                  ins[slot][c], lsems[slot]).wait()

    def issue_stores(t, slot):
        o = t * CHUNK
        for c in range(3):
            pltpu.async_copy(outs[slot][c],
                             out_hbm.at[pl.ds(bases[c] + o, CHUNK)], ssems[slot])

    def wait_stores(slot):
        for c in range(3):
            pltpu.make_async_copy(outs[slot][c],
                                  out_hbm.at[pl.ds(0, CHUNK)], ssems[slot]).wait()

    def compute_chunk(slot):
        @plsc.parallel_loop(0, CHUNK // L, 1, unroll=4)
        def vec_body(j):
            s = j * L
            rv = ins[slot][0][pl.ds(s, L)]
            gv = ins[slot][1][pl.ds(s, L)]
            bv = ins[slot][2][pl.ds(s, L)]
            ridx = rv * _INV_BS
            gidx = gv * _INV_BS
            bidx = bv * _INV_BS
            rid = jnp.minimum(ridx.astype(jnp.int32), DIM - 2)
            gid = jnp.minimum(gidx.astype(jnp.int32), DIM - 2)
            bid = jnp.minimum(bidx.astype(jnp.int32), DIM - 2)
            rd = ridx - rid.astype(jnp.float32)
            gd = gidx - gid.astype(jnp.float32)
            bd = bidx - bid.astype(jnp.float32)
            id000 = rid + gid * DIM + bid * (DIM * DIM)
            ids = tuple(id000 + o for o in _OFFS[1:])
            r1 = jnp.float32(1) - rd
            g1 = jnp.float32(1) - gd
            b1 = jnp.float32(1) - bd
            pg0 = g1 * b1
            pg1 = gd * b1
            pg2 = g1 * bd
            pg3 = gd * bd
            ws = (r1 * pg0, rd * pg0, r1 * pg1, rd * pg1,
                  r1 * pg2, rd * pg2, r1 * pg3, rd * pg3)
            all_ids = (id000,) + ids
            for c in range(3):
                row = tbls[c]
                terms = [ws[k] * plsc.load_gather(row, [all_ids[k]])
                         for k in range(8)]
                while len(terms) > 1:
                    terms = [terms[i] + terms[i + 1]
                             for i in range(0, len(terms), 2)]
                outs[slot][c][pl.ds(s, L)] = terms[0]

    issue_loads(0, 0)
    for cp in tbl_cps:
        cp.wait()

    def pair_body(t, carry):
        t0 = t * 2
        wait_loads(0)
        issue_loads(t0 + 1, 1)

        @pl.when(t > 0)
        def _():
            wait_stores(0)

        compute_chunk(0)
        issue_stores(t0, 0)
        wait_loads(1)

        @pl.when(t + 1 < NCHUNKS // 2)
        def _():
            issue_loads(t0 + 2, 0)

        @pl.when(t > 0)
        def _():
            wait_stores(1)

        compute_chunk(1)
        issue_stores(t0 + 1, 1)
        return carry

    lax.fori_loop(0, NCHUNKS // 2, pair_body, 0, unroll=False)
    wait_stores(0)
    wait_stores(1)


@jax.jit
def _run(lut_rows, xf):
    mesh = plsc.VectorSubcoreMesh(core_axis_name="c", subcore_axis_name="s")
    f = pl.kernel(
        _body,
        out_type=jax.ShapeDtypeStruct((NBATCH * 3 * NPX,), jnp.float32),
        mesh=mesh,
        compiler_params=pltpu.CompilerParams(needs_layout_passes=False),
        scratch_types=[
            pltpu.VMEM((ROW,), jnp.float32),
            pltpu.VMEM((ROW,), jnp.float32),
            pltpu.VMEM((ROW,), jnp.float32),
        ] + [pltpu.VMEM((CHUNK,), jnp.float32)] * 12 + [
            pltpu.SemaphoreType.DMA,
            pltpu.SemaphoreType.DMA,
            pltpu.SemaphoreType.DMA,
            pltpu.SemaphoreType.DMA,
            pltpu.SemaphoreType.DMA,
        ],
    )
    return f(lut_rows, xf)


def kernel(lut, x):
    lut_rows = jnp.pad(lut.reshape(3, SHIFT), ((0, 0), (0, ROW - SHIFT)))
    out_flat = _run(lut_rows.reshape(-1), x.reshape(-1))
    return out_flat.reshape(x.shape)


# speedup vs baseline: 1.5362x; 1.5362x over previous
import jax
import jax.numpy as jnp
import numpy as np
from jax import lax
from jax.experimental import pallas as pl
from jax.experimental.pallas import tpu as pltpu
from jax.experimental.pallas import tpu_sc as plsc

DIM = 33
SHIFT = DIM ** 3
ROW = 35944
NPX = 512 * 512
NBATCH = 4
CHUNK = 1024
L = 16
NWORKERS = 32
PX_PER_W = NBATCH * NPX // NWORKERS
NCHUNKS = PX_PER_W // CHUNK

_INV_BS = np.float32(1.0 / (1.000001 / (DIM - 1)))
_OFFS = (0, 1, DIM, DIM + 1, DIM * DIM, DIM * DIM + 1,
         DIM * DIM + DIM, DIM * DIM + DIM + 1)
_LOAD_BYTES = 3 * CHUNK * 4


def _body(lut_hbm, x_hbm, out_hbm, tbl0, tbl1, tbl2,
          rin0, gin0, bin0, rin1, gin1, bin1,
          rout0, gout0, bout0, rout1, gout1, bout1,
          tsem, lsem0, lsem1, ssem0, ssem1):
    ins = ((rin0, gin0, bin0), (rin1, gin1, bin1))
    outs = ((rout0, gout0, bout0), (rout1, gout1, bout1))
    lsems = (lsem0, lsem1)
    ssems = (ssem0, ssem1)

    nc = lax.axis_size("c")
    wid = lax.axis_index("s") * nc + lax.axis_index("c")

    w_per_batch = NPX // PX_PER_W
    batch = wid // w_per_batch
    plane_off = (wid % w_per_batch) * PX_PER_W
    bases = ((batch * 3 + 0) * NPX + plane_off,
             (batch * 3 + 1) * NPX + plane_off,
             (batch * 3 + 2) * NPX + plane_off)

    tbls = (tbl0, tbl1, tbl2)
    tbl_cps = [pltpu.async_copy(lut_hbm.at[pl.ds(c * ROW, ROW)],
                                tbls[c], tsem) for c in range(3)]

    def issue_loads(t, slot):
        o = t * CHUNK
        for c in range(3):
            pltpu.async_copy(x_hbm.at[pl.ds(bases[c] + o, CHUNK)],
                             ins[slot][c], lsems[slot])

    def wait_loads(slot):
        for c in range(3):
            pltpu.make_async_copy(x_hbm.at[pl.ds(0, CHUNK)],
                                  ins[slot][c], lsems[slot]).wait()

    def issue_stores(t, slot):
        o = t * CHUNK
        for c in range(3):
            pltpu.async_copy(outs[slot][c],
                             out_hbm.at[pl.ds(bases[c] + o, CHUNK)], ssems[slot])

    def wait_stores(slot):
        for c in range(3):
            pltpu.make_async_copy(outs[slot][c],
                                  out_hbm.at[pl.ds(0, CHUNK)], ssems[slot]).wait()

    def compute_chunk(slot):
        def vec_body(j, carry):
            s = j * L
            rv = ins[slot][0][pl.ds(s, L)]
            gv = ins[slot][1][pl.ds(s, L)]
            bv = ins[slot][2][pl.ds(s, L)]
            ridx = rv * _INV_BS
            gidx = gv * _INV_BS
            bidx = bv * _INV_BS
            rid = jnp.minimum(ridx.astype(jnp.int32), DIM - 2)
            gid = jnp.minimum(gidx.astype(jnp.int32), DIM - 2)
            bid = jnp.minimum(bidx.astype(jnp.int32), DIM - 2)
            rd = ridx - rid.astype(jnp.float32)
            gd = gidx - gid.astype(jnp.float32)
            bd = bidx - bid.astype(jnp.float32)
            id000 = rid + gid * DIM + bid * (DIM * DIM)
            ids = tuple(id000 + o for o in _OFFS[1:])
            r1 = jnp.float32(1) - rd
            g1 = jnp.float32(1) - gd
            b1 = jnp.float32(1) - bd
            pg0 = g1 * b1
            pg1 = gd * b1
            pg2 = g1 * bd
            pg3 = gd * bd
            ws = (r1 * pg0, rd * pg0, r1 * pg1, rd * pg1,
                  r1 * pg2, rd * pg2, r1 * pg3, rd * pg3)
            all_ids = (id000,) + ids
            gathered = [[plsc.load_gather(tbls[c], [all_ids[k]])
                         for k in range(8)] for c in range(3)]
            results = []
            for c in range(3):
                terms = [ws[k] * gathered[c][k] for k in range(8)]
                while len(terms) > 1:
                    terms = [terms[i] + terms[i + 1]
                             for i in range(0, len(terms), 2)]
                results.append(terms[0])
            for c in range(3):
                outs[slot][c][pl.ds(s, L)] = results[c]
            return carry

        lax.fori_loop(0, CHUNK // L, vec_body, 0, unroll=2)

    issue_loads(0, 0)
    for cp in tbl_cps:
        cp.wait()

    def pair_body(t, carry):
        t0 = t * 2
        wait_loads(0)
        issue_loads(t0 + 1, 1)

        @pl.when(t > 0)
        def _():
            wait_stores(0)

        compute_chunk(0)
        issue_stores(t0, 0)
        wait_loads(1)

        @pl.when(t + 1 < NCHUNKS // 2)
        def _():
            issue_loads(t0 + 2, 0)

        @pl.when(t > 0)
        def _():
            wait_stores(1)

        compute_chunk(1)
        issue_stores(t0 + 1, 1)
        return carry

    lax.fori_loop(0, NCHUNKS // 2, pair_body, 0, unroll=False)
    wait_stores(0)
    wait_stores(1)


@jax.jit
def _run(lut_rows, xf):
    mesh = plsc.VectorSubcoreMesh(core_axis_name="c", subcore_axis_name="s")
    f = pl.kernel(
        _body,
        out_type=jax.ShapeDtypeStruct((NBATCH * 3 * NPX,), jnp.float32),
        mesh=mesh,
        compiler_params=pltpu.CompilerParams(needs_layout_passes=False),
        scratch_types=[
            pltpu.VMEM((ROW,), jnp.float32),
            pltpu.VMEM((ROW,), jnp.float32),
            pltpu.VMEM((ROW,), jnp.float32),
        ] + [pltpu.VMEM((CHUNK,), jnp.float32)] * 12 + [
            pltpu.SemaphoreType.DMA,
            pltpu.SemaphoreType.DMA,
            pltpu.SemaphoreType.DMA,
            pltpu.SemaphoreType.DMA,
            pltpu.SemaphoreType.DMA,
        ],
    )
    return f(lut_rows, xf)


def kernel(lut, x):
    lut_rows = jnp.pad(lut.reshape(3, SHIFT), ((0, 0), (0, ROW - SHIFT)))
    out_flat = _run(lut_rows.reshape(-1), x.reshape(-1))
    return out_flat.reshape(x.shape)


# speedup vs baseline: 1.8267x; 1.1891x over previous
import jax
import jax.numpy as jnp
import numpy as np
from jax import lax
from jax.experimental import pallas as pl
from jax.experimental.pallas import tpu as pltpu
from jax.experimental.pallas import tpu_sc as plsc

DIM = 33
SHIFT = DIM ** 3
ROW = 35944
NPX = 512 * 512
NBATCH = 4
CHUNK = 1024
L = 16
NWORKERS = 32
PX_PER_W = NBATCH * NPX // NWORKERS
NCHUNKS = PX_PER_W // CHUNK

_INV_BS = np.float32(1.0 / (1.000001 / (DIM - 1)))
_OFFS = (0, 1, DIM, DIM + 1, DIM * DIM, DIM * DIM + 1,
         DIM * DIM + DIM, DIM * DIM + DIM + 1)
_LOAD_BYTES = 3 * CHUNK * 4


def _body(lut_hbm, x_hbm, out_hbm, tbl0, tbl1, tbl2,
          rin0, gin0, bin0, rin1, gin1, bin1,
          rout0, gout0, bout0, rout1, gout1, bout1,
          tsem, lsem0, lsem1, ssem0, ssem1):
    ins = ((rin0, gin0, bin0), (rin1, gin1, bin1))
    outs = ((rout0, gout0, bout0), (rout1, gout1, bout1))
    lsems = (lsem0, lsem1)
    ssems = (ssem0, ssem1)

    nc = lax.axis_size("c")
    wid = lax.axis_index("s") * nc + lax.axis_index("c")

    w_per_batch = NPX // PX_PER_W
    batch = wid // w_per_batch
    plane_off = (wid % w_per_batch) * PX_PER_W
    bases = ((batch * 3 + 0) * NPX + plane_off,
             (batch * 3 + 1) * NPX + plane_off,
             (batch * 3 + 2) * NPX + plane_off)

    tbls = (tbl0, tbl1, tbl2)
    tbl_cps = [pltpu.async_copy(lut_hbm.at[pl.ds(c * ROW, ROW)],
                                tbls[c], tsem) for c in range(3)]

    def issue_loads(t, slot):
        o = t * CHUNK
        for c in range(3):
            pltpu.async_copy(x_hbm.at[pl.ds(bases[c] + o, CHUNK)],
                             ins[slot][c], lsems[slot])

    def wait_loads(slot):
        for c in range(3):
            pltpu.make_async_copy(x_hbm.at[pl.ds(0, CHUNK)],
                                  ins[slot][c], lsems[slot]).wait()

    def issue_stores(t, slot):
        o = t * CHUNK
        for c in range(3):
            pltpu.async_copy(outs[slot][c],
                             out_hbm.at[pl.ds(bases[c] + o, CHUNK)], ssems[slot])

    def wait_stores(slot):
        for c in range(3):
            pltpu.make_async_copy(outs[slot][c],
                                  out_hbm.at[pl.ds(0, CHUNK)], ssems[slot]).wait()

    def compute_chunk(slot):
        def idx_weights(s):
            rv = ins[slot][0][pl.ds(s, L)]
            gv = ins[slot][1][pl.ds(s, L)]
            bv = ins[slot][2][pl.ds(s, L)]
            ridx = rv * _INV_BS
            gidx = gv * _INV_BS
            bidx = bv * _INV_BS
            rid = jnp.minimum(ridx.astype(jnp.int32), DIM - 2)
            gid = jnp.minimum(gidx.astype(jnp.int32), DIM - 2)
            bid = jnp.minimum(bidx.astype(jnp.int32), DIM - 2)
            rd = ridx - rid.astype(jnp.float32)
            gd = gidx - gid.astype(jnp.float32)
            bd = bidx - bid.astype(jnp.float32)
            id000 = rid + gid * DIM + bid * (DIM * DIM)
            all_ids = (id000,) + tuple(id000 + o for o in _OFFS[1:])
            r1 = jnp.float32(1) - rd
            g1 = jnp.float32(1) - gd
            b1 = jnp.float32(1) - bd
            pg0 = g1 * b1
            pg1 = gd * b1
            pg2 = g1 * bd
            pg3 = gd * bd
            ws = (r1 * pg0, rd * pg0, r1 * pg1, rd * pg1,
                  r1 * pg2, rd * pg2, r1 * pg3, rd * pg3)
            return all_ids, ws

        def gather_all(all_ids):
            return [[plsc.load_gather(tbls[c], [all_ids[k]])
                     for k in range(8)] for c in range(3)]

        def combine(gathered, ws):
            results = []
            for c in range(3):
                terms = [ws[k] * gathered[c][k] for k in range(8)]
                while len(terms) > 1:
                    terms = [terms[i] + terms[i + 1]
                             for i in range(0, len(terms), 2)]
                results.append(terms[0])
            return results

        def vec_body(j, carry):
            s0 = j * (2 * L)
            s1 = s0 + L
            ids0, ws0 = idx_weights(s0)
            ids1, ws1 = idx_weights(s1)
            g0 = gather_all(ids0)
            g1 = gather_all(ids1)
            res0 = combine(g0, ws0)
            res1 = combine(g1, ws1)
            for c in range(3):
                outs[slot][c][pl.ds(s0, L)] = res0[c]
            for c in range(3):
                outs[slot][c][pl.ds(s1, L)] = res1[c]
            return carry

        lax.fori_loop(0, CHUNK // (2 * L), vec_body, 0, unroll=2)

    issue_loads(0, 0)
    for cp in tbl_cps:
        cp.wait()

    def pair_body(t, carry):
        t0 = t * 2
        wait_loads(0)
        issue_loads(t0 + 1, 1)

        @pl.when(t > 0)
        def _():
            wait_stores(0)

        compute_chunk(0)
        issue_stores(t0, 0)
        wait_loads(1)

        @pl.when(t + 1 < NCHUNKS // 2)
        def _():
            issue_loads(t0 + 2, 0)

        @pl.when(t > 0)
        def _():
            wait_stores(1)

        compute_chunk(1)
        issue_stores(t0 + 1, 1)
        return carry

    lax.fori_loop(0, NCHUNKS // 2, pair_body, 0, unroll=False)
    wait_stores(0)
    wait_stores(1)


@jax.jit
def _run(lut_rows, xf):
    mesh = plsc.VectorSubcoreMesh(core_axis_name="c", subcore_axis_name="s")
    f = pl.kernel(
        _body,
        out_type=jax.ShapeDtypeStruct((NBATCH * 3 * NPX,), jnp.float32),
        mesh=mesh,
        compiler_params=pltpu.CompilerParams(needs_layout_passes=False),
        scratch_types=[
            pltpu.VMEM((ROW,), jnp.float32),
            pltpu.VMEM((ROW,), jnp.float32),
            pltpu.VMEM((ROW,), jnp.float32),
        ] + [pltpu.VMEM((CHUNK,), jnp.float32)] * 12 + [
            pltpu.SemaphoreType.DMA,
            pltpu.SemaphoreType.DMA,
            pltpu.SemaphoreType.DMA,
            pltpu.SemaphoreType.DMA,
            pltpu.SemaphoreType.DMA,
        ],
    )
    return f(lut_rows, xf)


def kernel(lut, x):
    lut_rows = jnp.pad(lut.reshape(3, SHIFT), ((0, 0), (0, ROW - SHIFT)))
    out_flat = _run(lut_rows.reshape(-1), x.reshape(-1))
    return out_flat.reshape(x.shape)


# speedup vs baseline: 1.9713x; 1.0791x over previous
import jax
import jax.numpy as jnp
import numpy as np
from jax import lax
from jax.experimental import pallas as pl
from jax.experimental.pallas import tpu as pltpu
from jax.experimental.pallas import tpu_sc as plsc

DIM = 33
SHIFT = DIM ** 3
ROW = 35944
NPX = 512 * 512
NBATCH = 4
CHUNK = 1024
L = 16
NWORKERS = 32
PX_PER_W = NBATCH * NPX // NWORKERS
NCHUNKS = PX_PER_W // CHUNK

_INV_BS = np.float32(1.0 / (1.000001 / (DIM - 1)))
_OFFS = (0, 1, DIM, DIM + 1, DIM * DIM, DIM * DIM + 1,
         DIM * DIM + DIM, DIM * DIM + DIM + 1)
_LOAD_BYTES = 3 * CHUNK * 4


def _body(lut_hbm, x_hbm, out_hbm, tbl0, tbl1, tbl2,
          rin0, gin0, bin0, rin1, gin1, bin1,
          rout0, gout0, bout0, rout1, gout1, bout1,
          tsem, lsem0, lsem1, ssem0, ssem1):
    ins = ((rin0, gin0, bin0), (rin1, gin1, bin1))
    outs = ((rout0, gout0, bout0), (rout1, gout1, bout1))
    lsems = (lsem0, lsem1)
    ssems = (ssem0, ssem1)

    nc = lax.axis_size("c")
    wid = lax.axis_index("s") * nc + lax.axis_index("c")

    w_per_batch = NPX // PX_PER_W
    batch = wid // w_per_batch
    plane_off = (wid % w_per_batch) * PX_PER_W
    bases = ((batch * 3 + 0) * NPX + plane_off,
             (batch * 3 + 1) * NPX + plane_off,
             (batch * 3 + 2) * NPX + plane_off)

    tbls = (tbl0, tbl1, tbl2)
    tbl_cps = [pltpu.async_copy(lut_hbm.at[pl.ds(c * ROW, ROW)],
                                tbls[c], tsem) for c in range(3)]

    def issue_loads(t, slot):
        o = t * CHUNK
        for c in range(3):
            pltpu.async_copy(x_hbm.at[pl.ds(bases[c] + o, CHUNK)],
                             ins[slot][c], lsems[slot])

    def wait_loads(slot):
        for c in range(3):
            pltpu.make_async_copy(x_hbm.at[pl.ds(0, CHUNK)],
                                  ins[slot][c], lsems[slot]).wait()

    def issue_stores(t, slot):
        o = t * CHUNK
        for c in range(3):
            pltpu.async_copy(outs[slot][c],
                             out_hbm.at[pl.ds(bases[c] + o, CHUNK)], ssems[slot])

    def wait_stores(slot):
        for c in range(3):
            pltpu.make_async_copy(outs[slot][c],
                                  out_hbm.at[pl.ds(0, CHUNK)], ssems[slot]).wait()

    def compute_chunk(slot):
        def idx_weights(s):
            rv = ins[slot][0][pl.ds(s, L)]
            gv = ins[slot][1][pl.ds(s, L)]
            bv = ins[slot][2][pl.ds(s, L)]
            ridx = rv * _INV_BS
            gidx = gv * _INV_BS
            bidx = bv * _INV_BS
            rid = jnp.minimum(ridx.astype(jnp.int32), DIM - 2)
            gid = jnp.minimum(gidx.astype(jnp.int32), DIM - 2)
            bid = jnp.minimum(bidx.astype(jnp.int32), DIM - 2)
            rd = ridx - rid.astype(jnp.float32)
            gd = gidx - gid.astype(jnp.float32)
            bd = bidx - bid.astype(jnp.float32)
            id000 = rid + gid * DIM + bid * (DIM * DIM)
            all_ids = (id000,) + tuple(id000 + o for o in _OFFS[1:])
            r1 = jnp.float32(1) - rd
            g1 = jnp.float32(1) - gd
            b1 = jnp.float32(1) - bd
            pg0 = g1 * b1
            pg1 = gd * b1
            pg2 = g1 * bd
            pg3 = gd * bd
            ws = (r1 * pg0, rd * pg0, r1 * pg1, rd * pg1,
                  r1 * pg2, rd * pg2, r1 * pg3, rd * pg3)
            return all_ids, ws

        def gather_all(all_ids):
            return [[plsc.load_gather(tbls[c], [all_ids[k]])
                     for k in range(8)] for c in range(3)]

        def combine(gathered, ws):
            results = []
            for c in range(3):
                terms = [ws[k] * gathered[c][k] for k in range(8)]
                while len(terms) > 1:
                    terms = [terms[i] + terms[i + 1]
                             for i in range(0, len(terms), 2)]
                results.append(terms[0])
            return results

        def vec_body(j, carry):
            s0 = j * (4 * L)
            s1 = s0 + L
            s2 = s0 + 2 * L
            s3 = s0 + 3 * L
            ids0, ws0 = idx_weights(s0)
            ids1, ws1 = idx_weights(s1)
            g0 = gather_all(ids0)
            ids2, ws2 = idx_weights(s2)
            g1 = gather_all(ids1)
            res0 = combine(g0, ws0)
            ids3, ws3 = idx_weights(s3)
            g2 = gather_all(ids2)
            res1 = combine(g1, ws1)
            for c in range(3):
                outs[slot][c][pl.ds(s0, L)] = res0[c]
            g3 = gather_all(ids3)
            res2 = combine(g2, ws2)
            for c in range(3):
                outs[slot][c][pl.ds(s1, L)] = res1[c]
            res3 = combine(g3, ws3)
            for c in range(3):
                outs[slot][c][pl.ds(s2, L)] = res2[c]
            for c in range(3):
                outs[slot][c][pl.ds(s3, L)] = res3[c]
            return carry

        lax.fori_loop(0, CHUNK // (4 * L), vec_body, 0, unroll=1)

    issue_loads(0, 0)
    for cp in tbl_cps:
        cp.wait()

    def pair_body(t, carry):
        t0 = t * 2
        wait_loads(0)
        issue_loads(t0 + 1, 1)

        @pl.when(t > 0)
        def _():
            wait_stores(0)

        compute_chunk(0)
        issue_stores(t0, 0)
        wait_loads(1)

        @pl.when(t + 1 < NCHUNKS // 2)
        def _():
            issue_loads(t0 + 2, 0)

        @pl.when(t > 0)
        def _():
            wait_stores(1)

        compute_chunk(1)
        issue_stores(t0 + 1, 1)
        return carry

    lax.fori_loop(0, NCHUNKS // 2, pair_body, 0, unroll=False)
    wait_stores(0)
    wait_stores(1)


@jax.jit
def _run(lut_rows, xf):
    mesh = plsc.VectorSubcoreMesh(core_axis_name="c", subcore_axis_name="s")
    f = pl.kernel(
        _body,
        out_type=jax.ShapeDtypeStruct((NBATCH * 3 * NPX,), jnp.float32),
        mesh=mesh,
        compiler_params=pltpu.CompilerParams(needs_layout_passes=False),
        scratch_types=[
            pltpu.VMEM((ROW,), jnp.float32),
            pltpu.VMEM((ROW,), jnp.float32),
            pltpu.VMEM((ROW,), jnp.float32),
        ] + [pltpu.VMEM((CHUNK,), jnp.float32)] * 12 + [
            pltpu.SemaphoreType.DMA,
            pltpu.SemaphoreType.DMA,
            pltpu.SemaphoreType.DMA,
            pltpu.SemaphoreType.DMA,
            pltpu.SemaphoreType.DMA,
        ],
    )
    return f(lut_rows, xf)


def kernel(lut, x):
    lut_rows = jnp.pad(lut.reshape(3, SHIFT), ((0, 0), (0, ROW - SHIFT)))
    out_flat = _run(lut_rows.reshape(-1), x.reshape(-1))
    return out_flat.reshape(x.shape)


# speedup vs baseline: 2.3481x; 1.1911x over previous
import jax
import jax.numpy as jnp
import numpy as np
from jax import lax
from jax.experimental import pallas as pl
from jax.experimental.pallas import tpu as pltpu
from jax.experimental.pallas import tpu_sc as plsc

DIM = 33
SHIFT = DIM ** 3
ROW = 35944
NROWS = 512
NCOLS = 512
NBATCH = 4
CROWS = 4
CHUNK = CROWS * NCOLS
L = 16
NWORKERS = 32
ROWS_PER_W = NBATCH * NROWS // NWORKERS
NCHUNKS = ROWS_PER_W // CROWS
NSLOTS = 3

_INV_BS = np.float32(1.0 / (1.000001 / (DIM - 1)))
_OFFS = (0, 1, DIM, DIM + 1, DIM * DIM, DIM * DIM + 1,
         DIM * DIM + DIM, DIM * DIM + DIM + 1)


def _body(lut_hbm, x_hbm, out_hbm, tbl0, tbl1, tbl2,
          b00, b01, b02, b10, b11, b12, b20, b21, b22,
          tsem, sem0, sem1, sem2):
    bufs = ((b00, b01, b02), (b10, b11, b12), (b20, b21, b22))
    sems = (sem0, sem1, sem2)
    tbls = (tbl0, tbl1, tbl2)

    nc = lax.axis_size("c")
    wid = lax.axis_index("s") * nc + lax.axis_index("c")

    w_per_batch = NROWS // ROWS_PER_W
    batch = wid // w_per_batch
    row0 = (wid % w_per_batch) * ROWS_PER_W

    tbl_cps = [pltpu.async_copy(lut_hbm.at[pl.ds(c * ROW, ROW)],
                                tbls[c], tsem) for c in range(3)]

    def issue_loads(t, sl):
        r = row0 + t * CROWS
        return [pltpu.async_copy(x_hbm.at[batch, c, pl.ds(r, CROWS), :],
                                 bufs[sl][c], sems[sl]) for c in range(3)]

    def issue_stores(t, sl):
        r = row0 + t * CROWS
        return [pltpu.async_copy(bufs[sl][c],
                                 out_hbm.at[batch, c, pl.ds(r, CROWS), :],
                                 sems[sl]) for c in range(3)]

    def compute_chunk(sl):
        rin, gin, bin_ = bufs[sl]

        def idx_weights(i, s):
            rv = rin[i, pl.ds(s, L)]
            gv = gin[i, pl.ds(s, L)]
            bv = bin_[i, pl.ds(s, L)]
            ridx = rv * _INV_BS
            gidx = gv * _INV_BS
            bidx = bv * _INV_BS
            rid = jnp.minimum(ridx.astype(jnp.int32), DIM - 2)
            gid = jnp.minimum(gidx.astype(jnp.int32), DIM - 2)
            bid = jnp.minimum(bidx.astype(jnp.int32), DIM - 2)
            rd = ridx - rid.astype(jnp.float32)
            gd = gidx - gid.astype(jnp.float32)
            bd = bidx - bid.astype(jnp.float32)
            id000 = rid + gid * DIM + bid * (DIM * DIM)
            all_ids = (id000,) + tuple(id000 + o for o in _OFFS[1:])
            r1 = jnp.float32(1) - rd
            g1 = jnp.float32(1) - gd
            b1 = jnp.float32(1) - bd
            pg0 = g1 * b1
            pg1 = gd * b1
            pg2 = g1 * bd
            pg3 = gd * bd
            ws = (r1 * pg0, rd * pg0, r1 * pg1, rd * pg1,
                  r1 * pg2, rd * pg2, r1 * pg3, rd * pg3)
            return all_ids, ws

        def gather_all(all_ids):
            return [[plsc.load_gather(tbls[c], [all_ids[k]])
                     for k in range(8)] for c in range(3)]

        def combine(gathered, ws):
            results = []
            for c in range(3):
                terms = [ws[k] * gathered[c][k] for k in range(8)]
                while len(terms) > 1:
                    terms = [terms[i] + terms[i + 1]
                             for i in range(0, len(terms), 2)]
                results.append(terms[0])
            return results

        def store(i, s, res):
            rin[i, pl.ds(s, L)] = res[0]
            gin[i, pl.ds(s, L)] = res[1]
            bin_[i, pl.ds(s, L)] = res[2]

        def vec_body(j, carry):
            i = j // 8
            cb = (j % 8) * (4 * L)
            s0, s1, s2, s3 = cb, cb + L, cb + 2 * L, cb + 3 * L
            ids0, ws0 = idx_weights(i, s0)
            ids1, ws1 = idx_weights(i, s1)
            g0 = gather_all(ids0)
            ids2, ws2 = idx_weights(i, s2)
            g1 = gather_all(ids1)
            res0 = combine(g0, ws0)
            ids3, ws3 = idx_weights(i, s3)
            g2 = gather_all(ids2)
            res1 = combine(g1, ws1)
            store(i, s0, res0)
            g3 = gather_all(ids3)
            res2 = combine(g2, ws2)
            store(i, s1, res1)
            res3 = combine(g3, ws3)
            store(i, s2, res2)
            store(i, s3, res3)
            return carry

        lax.fori_loop(0, CHUNK // (4 * L), vec_body, 0, unroll=1)

    load_cps = [None] * NSLOTS
    store_cps = [None] * NSLOTS
    load_cps[0] = issue_loads(0, 0)
    load_cps[1] = issue_loads(1, 1)
    for cp in tbl_cps:
        cp.wait()

    for t in range(NCHUNKS):
        sl = t % NSLOTS
        for cp in load_cps[sl]:
            cp.wait()
        compute_chunk(sl)
        store_cps[sl] = issue_stores(t, sl)
        if t + 2 < NCHUNKS:
            nsl = (t + 2) % NSLOTS
            if store_cps[nsl] is not None:
                for cp in store_cps[nsl]:
                    cp.wait()
                store_cps[nsl] = None
            load_cps[nsl] = issue_loads(t + 2, nsl)
    for cps in store_cps:
        if cps is not None:
            for cp in cps:
                cp.wait()


@jax.jit
def _run(lut_rows, x):
    mesh = plsc.VectorSubcoreMesh(core_axis_name="c", subcore_axis_name="s")
    f = pl.kernel(
        _body,
        out_type=jax.ShapeDtypeStruct((NBATCH, 3, NROWS, NCOLS), jnp.float32),
        mesh=mesh,
        compiler_params=pltpu.CompilerParams(
            needs_layout_passes=False, use_tc_tiling_on_sc=True),
        scratch_types=[
            pltpu.VMEM((ROW,), jnp.float32),
            pltpu.VMEM((ROW,), jnp.float32),
            pltpu.VMEM((ROW,), jnp.float32),
        ] + [pltpu.VMEM((CROWS, NCOLS), jnp.float32)] * 9 + [
            pltpu.SemaphoreType.DMA,
            pltpu.SemaphoreType.DMA,
            pltpu.SemaphoreType.DMA,
            pltpu.SemaphoreType.DMA,
        ],
    )
    return f(lut_rows, x)


def kernel(lut, x):
    lut_rows = jnp.pad(lut.reshape(3, SHIFT), ((0, 0), (0, ROW - SHIFT)))
    return _run(lut_rows.reshape(-1), x)


# speedup vs baseline: 2.4106x; 1.0266x over previous
import jax
import jax.numpy as jnp
import numpy as np
from jax import lax
from jax.experimental import pallas as pl
from jax.experimental.pallas import tpu as pltpu
from jax.experimental.pallas import tpu_sc as plsc

DIM = 33
SHIFT = DIM ** 3
ROW = 35944
NROWS = 512
NCOLS = 512
NBATCH = 4
CROWS = 4
CHUNK = CROWS * NCOLS
L = 16
NWORKERS = 32
ROWS_PER_W = NBATCH * NROWS // NWORKERS
NCHUNKS = ROWS_PER_W // CROWS
NSLOTS = 3

_INV_BS = np.float32(1.0 / (1.000001 / (DIM - 1)))
_OFFS = (0, 1, DIM, DIM + 1, DIM * DIM, DIM * DIM + 1,
         DIM * DIM + DIM, DIM * DIM + DIM + 1)


def _body(lut_hbm, x_hbm, out_hbm, tbl0, tbl1, tbl2,
          b00, b01, b02, b10, b11, b12, b20, b21, b22,
          tsem, sem0, sem1, sem2):
    bufs = ((b00, b01, b02), (b10, b11, b12), (b20, b21, b22))
    sems = (sem0, sem1, sem2)
    tbls = (tbl0, tbl1, tbl2)

    nc = lax.axis_size("c")
    wid = lax.axis_index("s") * nc + lax.axis_index("c")

    w_per_batch = NROWS // ROWS_PER_W
    batch = wid // w_per_batch
    row0 = (wid % w_per_batch) * ROWS_PER_W

    tbl_cps = [pltpu.async_copy(lut_hbm.at[pl.ds(c * ROW, ROW)],
                                tbls[c], tsem) for c in range(3)]

    def issue_loads(t, sl):
        r = row0 + t * CROWS
        for c in range(3):
            pltpu.async_copy(x_hbm.at[batch, c, pl.ds(r, CROWS), :],
                             bufs[sl][c], sems[sl])

    def wait_loads(sl):
        for c in range(3):
            pltpu.make_async_copy(x_hbm.at[0, 0, pl.ds(0, CROWS), :],
                                  bufs[sl][c], sems[sl]).wait()

    def issue_stores(t, sl):
        r = row0 + t * CROWS
        for c in range(3):
            pltpu.async_copy(bufs[sl][c],
                             out_hbm.at[batch, c, pl.ds(r, CROWS), :],
                             sems[sl])

    def wait_stores(sl):
        for c in range(3):
            pltpu.make_async_copy(bufs[sl][c],
                                  out_hbm.at[0, 0, pl.ds(0, CROWS), :],
                                  sems[sl]).wait()

    def compute_chunk(sl):
        rin, gin, bin_ = bufs[sl]

        def idx_weights(i, s):
            rv = rin[i, pl.ds(s, L)]
            gv = gin[i, pl.ds(s, L)]
            bv = bin_[i, pl.ds(s, L)]
            ridx = rv * _INV_BS
            gidx = gv * _INV_BS
            bidx = bv * _INV_BS
            rid = jnp.minimum(ridx.astype(jnp.int32), DIM - 2)
            gid = jnp.minimum(gidx.astype(jnp.int32), DIM - 2)
            bid = jnp.minimum(bidx.astype(jnp.int32), DIM - 2)
            rd = ridx - rid.astype(jnp.float32)
            gd = gidx - gid.astype(jnp.float32)
            bd = bidx - bid.astype(jnp.float32)
            id000 = rid + gid * DIM + bid * (DIM * DIM)
            all_ids = (id000,) + tuple(id000 + o for o in _OFFS[1:])
            r1 = jnp.float32(1) - rd
            g1 = jnp.float32(1) - gd
            b1 = jnp.float32(1) - bd
            pg0 = g1 * b1
            pg1 = gd * b1
            pg2 = g1 * bd
            pg3 = gd * bd
            ws = (r1 * pg0, rd * pg0, r1 * pg1, rd * pg1,
                  r1 * pg2, rd * pg2, r1 * pg3, rd * pg3)
            return all_ids, ws

        def gather_all(all_ids):
            return [[plsc.load_gather(tbls[c], [all_ids[k]])
                     for k in range(8)] for c in range(3)]

        def combine(gathered, ws):
            results = []
            for c in range(3):
                terms = [ws[k] * gathered[c][k] for k in range(8)]
                while len(terms) > 1:
                    terms = [terms[i] + terms[i + 1]
                             for i in range(0, len(terms), 2)]
                results.append(terms[0])
            return results

        def store(i, s, res):
            rin[i, pl.ds(s, L)] = res[0]
            gin[i, pl.ds(s, L)] = res[1]
            bin_[i, pl.ds(s, L)] = res[2]

        NV = 4

        def vec_body(j, carry):
            i = j // (NCOLS // (NV * L))
            cb = (j % (NCOLS // (NV * L))) * (NV * L)
            ss = [cb + k * L for k in range(NV)]
            iw = [None] * NV
            g = [None] * NV
            res = [None] * NV
            iw[0] = idx_weights(i, ss[0])
            iw[1] = idx_weights(i, ss[1])
            g[0] = gather_all(iw[0][0])
            for k in range(NV):
                if k + 2 < NV:
                    iw[k + 2] = idx_weights(i, ss[k + 2])
                if k + 1 < NV:
                    g[k + 1] = gather_all(iw[k + 1][0])
                res[k] = combine(g[k], iw[k][1])
                g[k] = None
                if k >= 1:
                    store(i, ss[k - 1], res[k - 1])
            store(i, ss[NV - 1], res[NV - 1])
            return carry

        lax.fori_loop(0, CHUNK // (NV * L), vec_body, 0, unroll=1)

    issue_loads(0, 0)
    issue_loads(1, 1)
    for cp in tbl_cps:
        cp.wait()

    def chunk_step(t, sl, prefetch):
        nsl = (sl + 2) % NSLOTS
        wait_loads(sl)
        compute_chunk(sl)
        issue_stores(t, sl)

        @pl.when(t > 0)
        def _():
            wait_stores(nsl)

        if prefetch:
            @pl.when(t + 2 < NCHUNKS)
            def _():
                issue_loads(t + 2, nsl)

    def triple_body(q, carry):
        for k in range(NSLOTS):
            chunk_step(q * NSLOTS + k, k, True)
        return carry

    lax.fori_loop(0, NCHUNKS // NSLOTS, triple_body, 0, unroll=1)
    chunk_step(NCHUNKS - 1, (NCHUNKS - 1) % NSLOTS, False)
    wait_stores((NCHUNKS - 1) % NSLOTS)


@jax.jit
def _run(lut_rows, x):
    mesh = plsc.VectorSubcoreMesh(core_axis_name="c", subcore_axis_name="s")
    f = pl.kernel(
        _body,
        out_type=jax.ShapeDtypeStruct((NBATCH, 3, NROWS, NCOLS), jnp.float32),
        mesh=mesh,
        compiler_params=pltpu.CompilerParams(
            needs_layout_passes=False, use_tc_tiling_on_sc=True),
        scratch_types=[
            pltpu.VMEM((ROW,), jnp.float32),
            pltpu.VMEM((ROW,), jnp.float32),
            pltpu.VMEM((ROW,), jnp.float32),
        ] + [pltpu.VMEM((CROWS, NCOLS), jnp.float32)] * 9 + [
            pltpu.SemaphoreType.DMA,
            pltpu.SemaphoreType.DMA,
            pltpu.SemaphoreType.DMA,
            pltpu.SemaphoreType.DMA,
        ],
    )
    return f(lut_rows, x)


def kernel(lut, x):
    lut_rows = jnp.pad(lut.reshape(3, SHIFT), ((0, 0), (0, ROW - SHIFT)))
    return _run(lut_rows.reshape(-1), x)


# speedup vs baseline: 2.4438x; 1.0138x over previous
import jax
import jax.numpy as jnp
import numpy as np
from jax import lax
from jax.experimental import pallas as pl
from jax.experimental.pallas import tpu as pltpu
from jax.experimental.pallas import tpu_sc as plsc

DIM = 33
SHIFT = DIM ** 3
ROW = 35944
NROWS = 512
NCOLS = 512
NBATCH = 4
CROWS = 4
CHUNK = CROWS * NCOLS
L = 16
NWORKERS = 32
ROWS_PER_W = NBATCH * NROWS // NWORKERS
NCHUNKS = ROWS_PER_W // CROWS
NSLOTS = 3

_INV_BS = np.float32(1.0 / (1.000001 / (DIM - 1)))
_OFFS = (0, 1, DIM, DIM + 1, DIM * DIM, DIM * DIM + 1,
         DIM * DIM + DIM, DIM * DIM + DIM + 1)


def _body(lut_hbm, x_hbm, out_hbm, tbl0, tbl1, tbl2,
          b00, b01, b02, b10, b11, b12, b20, b21, b22,
          tsem, sem0, sem1, sem2):
    bufs = ((b00, b01, b02), (b10, b11, b12), (b20, b21, b22))
    sems = (sem0, sem1, sem2)
    tbls = (tbl0, tbl1, tbl2)

    nc = lax.axis_size("c")
    wid = lax.axis_index("s") * nc + lax.axis_index("c")

    w_per_batch = NROWS // ROWS_PER_W
    batch = wid // w_per_batch
    row0 = (wid % w_per_batch) * ROWS_PER_W

    tbl_cps = [pltpu.async_copy(lut_hbm.at[pl.ds(c * ROW, ROW)],
                                tbls[c], tsem) for c in range(3)]

    def issue_loads(t, sl):
        r = row0 + t * CROWS
        for c in range(3):
            pltpu.async_copy(x_hbm.at[batch, c, pl.ds(r, CROWS), :],
                             bufs[sl][c], sems[sl])

    def wait_loads(sl):
        for c in range(3):
            pltpu.make_async_copy(x_hbm.at[0, 0, pl.ds(0, CROWS), :],
                                  bufs[sl][c], sems[sl]).wait()

    def issue_stores(t, sl):
        r = row0 + t * CROWS
        for c in range(3):
            pltpu.async_copy(bufs[sl][c],
                             out_hbm.at[batch, c, pl.ds(r, CROWS), :],
                             sems[sl])

    def wait_stores(sl):
        for c in range(3):
            pltpu.make_async_copy(bufs[sl][c],
                                  out_hbm.at[0, 0, pl.ds(0, CROWS), :],
                                  sems[sl]).wait()

    def compute_chunk(sl):
        rin, gin, bin_ = bufs[sl]

        def idx_weights(i, s):
            rv = rin[i, pl.ds(s, L)]
            gv = gin[i, pl.ds(s, L)]
            bv = bin_[i, pl.ds(s, L)]
            ridx = rv * _INV_BS
            gidx = gv * _INV_BS
            bidx = bv * _INV_BS
            rid = jnp.minimum(ridx.astype(jnp.int32), DIM - 2)
            gid = jnp.minimum(gidx.astype(jnp.int32), DIM - 2)
            bid = jnp.minimum(bidx.astype(jnp.int32), DIM - 2)
            rd = ridx - rid.astype(jnp.float32)
            gd = gidx - gid.astype(jnp.float32)
            bd = bidx - bid.astype(jnp.float32)
            id000 = rid + gid * DIM + bid * (DIM * DIM)
            all_ids = (id000,) + tuple(id000 + o for o in _OFFS[1:])
            r1 = jnp.float32(1) - rd
            g1 = jnp.float32(1) - gd
            b1 = jnp.float32(1) - bd
            pg0 = g1 * b1
            pg1 = gd * b1
            pg2 = g1 * bd
            pg3 = gd * bd
            ws = (r1 * pg0, rd * pg0, r1 * pg1, rd * pg1,
                  r1 * pg2, rd * pg2, r1 * pg3, rd * pg3)
            return all_ids, ws

        def gather_all(all_ids):
            return [[plsc.load_gather(tbls[c], [all_ids[k]])
                     for k in range(8)] for c in range(3)]

        def combine(gathered, ws):
            results = []
            for c in range(3):
                terms = [ws[k] * gathered[c][k] for k in range(8)]
                while len(terms) > 1:
                    terms = [terms[i] + terms[i + 1]
                             for i in range(0, len(terms), 2)]
                results.append(terms[0])
            return results

        def store(i, s, res):
            rin[i, pl.ds(s, L)] = res[0]
            gin[i, pl.ds(s, L)] = res[1]
            bin_[i, pl.ds(s, L)] = res[2]

        NV = 4

        def vec_body(j, carry):
            i = j // (NCOLS // (NV * L))
            cb = (j % (NCOLS // (NV * L))) * (NV * L)
            ss = [cb + k * L for k in range(NV)]
            iw = [None] * NV
            g = [None] * NV
            res = [None] * NV
            iw[0] = idx_weights(i, ss[0])
            iw[1] = idx_weights(i, ss[1])
            g[0] = gather_all(iw[0][0])
            for k in range(NV):
                if k + 2 < NV:
                    iw[k + 2] = idx_weights(i, ss[k + 2])
                if k + 1 < NV:
                    g[k + 1] = gather_all(iw[k + 1][0])
                res[k] = combine(g[k], iw[k][1])
                g[k] = None
                if k >= 1:
                    store(i, ss[k - 1], res[k - 1])
            store(i, ss[NV - 1], res[NV - 1])
            return carry

        lax.fori_loop(0, CHUNK // (NV * L), vec_body, 0, unroll=2)

    issue_loads(0, 0)
    issue_loads(1, 1)
    for cp in tbl_cps:
        cp.wait()

    def chunk_step(t, sl, prefetch):
        nsl = (sl + 2) % NSLOTS
        wait_loads(sl)
        compute_chunk(sl)
        issue_stores(t, sl)

        @pl.when(t > 0)
        def _():
            wait_stores(nsl)

        if prefetch:
            @pl.when(t + 2 < NCHUNKS)
            def _():
                issue_loads(t + 2, nsl)

    def triple_body(q, carry):
        for k in range(NSLOTS):
            chunk_step(q * NSLOTS + k, k, True)
        return carry

    lax.fori_loop(0, NCHUNKS // NSLOTS, triple_body, 0, unroll=1)
    chunk_step(NCHUNKS - 1, (NCHUNKS - 1) % NSLOTS, False)
    wait_stores((NCHUNKS - 1) % NSLOTS)


@jax.jit
def _run(lut_rows, x):
    mesh = plsc.VectorSubcoreMesh(core_axis_name="c", subcore_axis_name="s")
    f = pl.kernel(
        _body,
        out_type=jax.ShapeDtypeStruct((NBATCH, 3, NROWS, NCOLS), jnp.float32),
        mesh=mesh,
        compiler_params=pltpu.CompilerParams(
            needs_layout_passes=False, use_tc_tiling_on_sc=True),
        scratch_types=[
            pltpu.VMEM((ROW,), jnp.float32),
            pltpu.VMEM((ROW,), jnp.float32),
            pltpu.VMEM((ROW,), jnp.float32),
        ] + [pltpu.VMEM((CROWS, NCOLS), jnp.float32)] * 9 + [
            pltpu.SemaphoreType.DMA,
            pltpu.SemaphoreType.DMA,
            pltpu.SemaphoreType.DMA,
            pltpu.SemaphoreType.DMA,
        ],
    )
    return f(lut_rows, x)


def kernel(lut, x):
    lut_rows = jnp.pad(lut.reshape(3, SHIFT), ((0, 0), (0, ROW - SHIFT)))
    return _run(lut_rows.reshape(-1), x)
